# baseline (speedup 1.0000x reference)
import functools

import numpy as np

import jax
import jax.numpy as jnp
from jax import lax
from jax.experimental import pallas as pl
from jax.experimental.pallas import tpu as pltpu
from jax.experimental.pallas import tpu_sc as plsc

MAX_LEN = 200
VOCAB = 1000000
DIM = 64
BATCH = 4096

_INFO = plsc.get_sparse_core_info()
NC = _INFO.num_cores
NS = _INFO.num_subcores
LANES = _INFO.num_lanes
NW = NC * NS

NBB = 8
BB = BATCH // NBB
NLG = NW // NBB
LPG = MAX_LEN // NLG
QT = 128
NQ = BB // QT
DBLKS = DIM // LANES
PDIM = DIM


def _pos_encoding() -> np.ndarray:
    pos = np.arange(MAX_LEN, dtype=np.float64)[:, None]
    i = np.arange(0, DIM, 2, dtype=np.float64)[None, :]
    angle = pos / (10000.0 ** (2.0 * i / DIM))
    enc = np.zeros((MAX_LEN, DIM), dtype=np.float64)
    enc[:, 0::2] = np.sin(angle)
    enc[:, 1::2] = np.cos(angle)
    return enc.astype(np.float32)


TC = 320
NCH = VOCAB // TC
CPT = (NCH + NW - 1) // NW


def _make_transposer():
    mesh = plsc.VectorSubcoreMesh(core_axis_name="c", subcore_axis_name="s")
    TP = DIM + 1

    @functools.partial(
        pl.kernel,
        mesh=mesh,
        compiler_params=pltpu.CompilerParams(
            use_tc_tiling_on_sc=False, needs_layout_passes=False
        ),
        out_type=jax.ShapeDtypeStruct((VOCAB, DIM), jnp.float32),
        scratch_types=[
            pltpu.VMEM((DIM, TC), jnp.float32),
            pltpu.VMEM((DIM, TC), jnp.float32),
            pltpu.VMEM((TC, TP), jnp.float32),
            pltpu.VMEM((TC, TP), jnp.float32),
            pltpu.SemaphoreType.DMA,
            pltpu.SemaphoreType.DMA,
            pltpu.SemaphoreType.DMA,
            pltpu.SemaphoreType.DMA,
        ],
    )
    def k(tt_hbm, out_hbm, s0, s1, d0, d1, rs0, rs1, ws0, ws1):
        sbufs, dbufs = (s0, s1), (d0, d1)
        rss, wss = (rs0, rs1), (ws0, ws1)
        w = lax.axis_index("s") * NC + lax.axis_index("c")

        def ch_of(i):
            return i * NW + w

        def r_start(i, b):
            pltpu.async_copy(
                tt_hbm.at[:, pl.ds(ch_of(i) * TC, TC)], sbufs[b], rss[b]
            )

        def r_wait(i, b):
            pltpu.make_async_copy(
                tt_hbm.at[:, pl.ds(ch_of(i) * TC, TC)], sbufs[b], rss[b]
            ).wait()

        def w_start(i, b):
            pltpu.async_copy(
                dbufs[b].at[:, pl.ds(0, DIM)],
                out_hbm.at[pl.ds(ch_of(i) * TC, TC)], wss[b]
            )

        def w_wait(i, b):
            pltpu.make_async_copy(
                dbufs[b].at[:, pl.ds(0, DIM)],
                out_hbm.at[pl.ds(ch_of(i) * TC, TC)], wss[b]
            ).wait()

        r_start(0, 0)
        r_start(1, 1)
        c_iota = lax.iota(jnp.int32, LANES)
        c_rows = [c_iota + cb * LANES for cb in range(TC // LANES)]
        zeros16 = jnp.zeros((LANES,), jnp.int32)

        def body(p, carry):
            for b in range(2):
                i = 2 * p + b

                @pl.when((i >= 2) & (ch_of(i - 2) < NCH))
                def _(i=i, b=b):
                    w_wait(i - 2, b)

                @pl.when(ch_of(i) < NCH)
                def _(i=i, b=b):
                    r_wait(i, b)

                    @plsc.parallel_loop(0, DIM, unroll=4)
                    def _tr(d, b=b):
                        dcol = zeros16 + d
                        for cb in range(TC // LANES):
                            v = sbufs[b][d, pl.ds(cb * LANES, LANES)]
                            plsc.store_scatter(dbufs[b], [c_rows[cb], dcol], v)

                    @pl.when(ch_of(i + 2) < NCH)
                    def _(i=i, b=b):
                        r_start(i + 2, b)

                    w_start(i, b)
            return carry

        lax.fori_loop(0, CPT // 2, body, 0)

        for b in range(2):
            last_i = CPT - 2 + b

            @pl.when(ch_of(last_i) < NCH)
            def _(last_i=last_i, b=b):
                w_wait(last_i, b)

    return k


def _make_sc_kernel():
    mesh = plsc.VectorSubcoreMesh(core_axis_name="c", subcore_axis_name="s")

    @functools.partial(
        pl.kernel,
        mesh=mesh,
        compiler_params=pltpu.CompilerParams(
            use_tc_tiling_on_sc=False, needs_layout_passes=False
        ),
        out_type=jax.ShapeDtypeStruct((MAX_LEN, DIM, BATCH), jnp.float32),
        scratch_types=[
            pltpu.VMEM((LPG, BB), jnp.int32),
            pltpu.VMEM((QT, PDIM), jnp.float32),
            pltpu.VMEM((QT, PDIM), jnp.float32),
            pltpu.VMEM((DIM, BB + 1), jnp.float32),
            pltpu.VMEM((DIM, BB + 1), jnp.float32),
            pltpu.VMEM((LPG * DIM,), jnp.float32),
            pltpu.SemaphoreType.DMA,
            pltpu.SemaphoreType.DMA,
            pltpu.SemaphoreType.DMA,
            pltpu.SemaphoreType.DMA,
        ],
    )
    def k(idx_hbm, table_hbm, pe_hbm, out_hbm,
          idx_v, g0, g1, t0, t1, pe_v, gs0, gs1, os0, os1):
        gbufs, tbufs = (g0, g1), (t0, t1)
        gss, oss = (gs0, gs1), (os0, os1)
        w = lax.axis_index("s") * NC + lax.axis_index("c")
        grp = w // NBB
        b0 = (w % NBB) * BB
        pltpu.sync_copy(idx_hbm.at[w], idx_v)
        pltpu.sync_copy(pe_hbm.at[grp], pe_v)

        def g_start(li, q, qb):
            pltpu.async_copy(
                table_hbm.at[idx_v.at[li, pl.ds(q * QT, QT)]],
                gbufs[qb], gss[qb],
            )

        def g_wait(li, q, qb):
            pltpu.make_async_copy(
                table_hbm.at[idx_v.at[li, pl.ds(q * QT, QT)]],
                gbufs[qb], gss[qb],
            ).wait()

        g_start(0, 0, 0)
        g_start(0, 1, 1)
        d_iota = lax.iota(jnp.int32, LANES)
        zeros16 = jnp.zeros((LANES,), jnp.int32)
        d_rows = [d_iota + db * LANES for db in range(DBLKS)]

        def pair(p, carry):
            for j in range(2):
                li = 2 * p + j
                l = grp * LPG + li

                @pl.when(li >= 2)
                def _():
                    pltpu.make_async_copy(
                        tbufs[j].at[:, pl.ds(0, BB)],
                        out_hbm.at[l - 2, :, pl.ds(b0, BB)], oss[j]
                    ).wait()

                pe_vecs = [
                    pe_v[pl.ds(li * DIM + db * LANES, LANES)]
                    for db in range(DBLKS)
                ]

                for q in range(NQ):
                    qb = q % 2
                    g_wait(li, q, qb)

                    @plsc.parallel_loop(0, QT, unroll=8)
                    def _tr(r, j=j, q=q, qb=qb, pe_vecs=pe_vecs):
                        cols = zeros16 + (q * QT + r)
                        for db in range(DBLKS):
                            v = (gbufs[qb][r, pl.ds(db * LANES, LANES)]
                                 + pe_vecs[db])
                            plsc.store_scatter(
                                tbufs[j], [d_rows[db], cols], v
                            )

                    if q < 2:
                        g_start(li, q + 2, qb)
                    else:
                        @pl.when(li + 1 < LPG)
                        def _(li=li, q=q, qb=qb):
                            g_start(li + 1, q - 2, qb)

                pltpu.async_copy(
                    tbufs[j].at[:, pl.ds(0, BB)],
                    out_hbm.at[l, :, pl.ds(b0, BB)], oss[j]
                )
            return carry

        lax.fori_loop(0, LPG // 2, pair, 0)

        last = grp * LPG + LPG
        pltpu.make_async_copy(
            t0.at[:, pl.ds(0, BB)], out_hbm.at[last - 2, :, pl.ds(b0, BB)], os0
        ).wait()
        pltpu.make_async_copy(
            t1.at[:, pl.ds(0, BB)], out_hbm.at[last - 1, :, pl.ds(b0, BB)], os1
        ).wait()

    return k


_TRANSPOSER = _make_transposer()
_SC_KERNEL = _make_sc_kernel()


def kernel(batch, table):
    idx4 = (
        jnp.transpose(batch.astype(jnp.int32), (1, 0))
        .reshape(NLG, LPG, NBB, BB)
        .transpose(0, 2, 1, 3)
        .reshape(NW, LPG, BB)
    )
    table_rm = _TRANSPOSER(jnp.transpose(table, (1, 0)))
    pe = jnp.asarray(_pos_encoding()).reshape(NLG, LPG * DIM)
    out_t = _SC_KERNEL(idx4, table_rm, pe)
    return jnp.transpose(out_t, (2, 0, 1))


# speedup vs baseline: 3.7653x; 3.7653x over previous
import functools

import numpy as np

import jax
import jax.numpy as jnp
from jax import lax
from jax.experimental import pallas as pl
from jax.experimental.pallas import tpu as pltpu
from jax.experimental.pallas import tpu_sc as plsc

MAX_LEN = 200
VOCAB = 1000000
DIM = 64
BATCH = 4096

_INFO = plsc.get_sparse_core_info()
NC = _INFO.num_cores
NS = _INFO.num_subcores
LANES = _INFO.num_lanes
NW = NC * NS

NBB = 8
BB = BATCH // NBB
NLG = NW // NBB
LPG = MAX_LEN // NLG
QT = 128
NQ = BB // QT
DBLKS = DIM // LANES
PDIM = 80


def _pos_encoding() -> np.ndarray:
    pos = np.arange(MAX_LEN, dtype=np.float64)[:, None]
    i = np.arange(0, DIM, 2, dtype=np.float64)[None, :]
    angle = pos / (10000.0 ** (2.0 * i / DIM))
    enc = np.zeros((MAX_LEN, DIM), dtype=np.float64)
    enc[:, 0::2] = np.sin(angle)
    enc[:, 1::2] = np.cos(angle)
    return enc.astype(np.float32)


def _make_sc_kernel():
    mesh = plsc.VectorSubcoreMesh(core_axis_name="c", subcore_axis_name="s")

    @functools.partial(
        pl.kernel,
        mesh=mesh,
        compiler_params=pltpu.CompilerParams(
            use_tc_tiling_on_sc=False, needs_layout_passes=False
        ),
        out_type=jax.ShapeDtypeStruct((MAX_LEN, DIM, BATCH), jnp.float32),
        scratch_types=[
            pltpu.VMEM((LPG, BB), jnp.int32),
            pltpu.VMEM((QT, PDIM), jnp.float32),
            pltpu.VMEM((QT, PDIM), jnp.float32),
            pltpu.VMEM((DIM, BB + 1), jnp.float32),
            pltpu.VMEM((DIM, BB + 1), jnp.float32),
            pltpu.VMEM((LPG * DIM,), jnp.float32),
            pltpu.SemaphoreType.DMA,
            pltpu.SemaphoreType.DMA,
            pltpu.SemaphoreType.DMA,
            pltpu.SemaphoreType.DMA,
        ],
    )
    def k(idx_hbm, table_hbm, pe_hbm, out_hbm,
          idx_v, g0, g1, t0, t1, pe_v, gs0, gs1, os0, os1):
        gbufs, tbufs = (g0, g1), (t0, t1)
        gss, oss = (gs0, gs1), (os0, os1)
        w = lax.axis_index("s") * NC + lax.axis_index("c")
        grp = w // NBB
        b0 = (w % NBB) * BB
        pltpu.sync_copy(idx_hbm.at[w], idx_v)
        pltpu.sync_copy(pe_hbm.at[grp], pe_v)

        def g_start(li, q, qb):
            pltpu.async_copy(
                table_hbm.at[idx_v.at[li, pl.ds(q * QT, QT)]],
                gbufs[qb], gss[qb],
            )

        def g_wait(li, q, qb):
            pltpu.make_async_copy(
                table_hbm.at[idx_v.at[li, pl.ds(q * QT, QT)]],
                gbufs[qb], gss[qb],
            ).wait()

        g_start(0, 0, 0)
        g_start(0, 1, 1)
        d_iota = lax.iota(jnp.int32, LANES)
        zeros16 = jnp.zeros((LANES,), jnp.int32)
        d_rows = [d_iota + db * LANES for db in range(DBLKS)]

        def pair(p, carry):
            for j in range(2):
                li = 2 * p + j
                l = grp * LPG + li

                @pl.when(li >= 2)
                def _():
                    pltpu.make_async_copy(
                        tbufs[j].at[:, pl.ds(0, BB)],
                        out_hbm.at[l - 2, :, pl.ds(b0, BB)], oss[j]
                    ).wait()

                pe_vecs = [
                    pe_v[pl.ds(li * DIM + db * LANES, LANES)]
                    for db in range(DBLKS)
                ]

                for q in range(NQ):
                    qb = q % 2
                    g_wait(li, q, qb)

                    @plsc.parallel_loop(0, QT, unroll=8)
                    def _tr(r, j=j, q=q, qb=qb, pe_vecs=pe_vecs):
                        cols = zeros16 + (q * QT + r)
                        for db in range(DBLKS):
                            v = (gbufs[qb][r, pl.ds(db * LANES, LANES)]
                                 + pe_vecs[db])
                            plsc.store_scatter(
                                tbufs[j], [d_rows[db], cols], v
                            )

                    if q < 2:
                        g_start(li, q + 2, qb)
                    else:
                        @pl.when(li + 1 < LPG)
                        def _(li=li, q=q, qb=qb):
                            g_start(li + 1, q - 2, qb)

                pltpu.async_copy(
                    tbufs[j].at[:, pl.ds(0, BB)],
                    out_hbm.at[l, :, pl.ds(b0, BB)], oss[j]
                )
            return carry

        lax.fori_loop(0, LPG // 2, pair, 0)

        last = grp * LPG + LPG
        pltpu.make_async_copy(
            t0.at[:, pl.ds(0, BB)], out_hbm.at[last - 2, :, pl.ds(b0, BB)], os0
        ).wait()
        pltpu.make_async_copy(
            t1.at[:, pl.ds(0, BB)], out_hbm.at[last - 1, :, pl.ds(b0, BB)], os1
        ).wait()

    return k


_SC_KERNEL = _make_sc_kernel()


def kernel(batch, table):
    idx4 = (
        jnp.transpose(batch.astype(jnp.int32), (1, 0))
        .reshape(NLG, LPG, NBB, BB)
        .transpose(0, 2, 1, 3)
        .reshape(NW, LPG, BB)
    )
    table_p = jnp.pad(table, ((0, 0), (0, PDIM - DIM)))
    pe = jnp.asarray(_pos_encoding()).reshape(NLG, LPG * DIM)
    out_t = _SC_KERNEL(idx4, table_p, pe)
    return jnp.transpose(out_t, (2, 0, 1))


# speedup vs baseline: 5.6096x; 1.4898x over previous
import functools

import numpy as np

import jax
import jax.numpy as jnp
from jax import lax
from jax.experimental import pallas as pl
from jax.experimental.pallas import tpu as pltpu
from jax.experimental.pallas import tpu_sc as plsc

MAX_LEN = 200
VOCAB = 1000000
DIM = 64
BATCH = 4096

_INFO = plsc.get_sparse_core_info()
NC = _INFO.num_cores
NS = _INFO.num_subcores
LANES = _INFO.num_lanes
NW = NC * NS

NBB = 8
BB = BATCH // NBB
NLG = NW // NBB
LPG = MAX_LEN // NLG
QT = 128
NQ = BB // QT
DBLKS = DIM // LANES
PDIM = 2 * DIM


def _pos_encoding() -> np.ndarray:
    pos = np.arange(MAX_LEN, dtype=np.float64)[:, None]
    i = np.arange(0, DIM, 2, dtype=np.float64)[None, :]
    angle = pos / (10000.0 ** (2.0 * i / DIM))
    enc = np.zeros((MAX_LEN, DIM), dtype=np.float64)
    enc[:, 0::2] = np.sin(angle)
    enc[:, 1::2] = np.cos(angle)
    return enc.astype(np.float32)


def _make_sc_kernel():
    mesh = plsc.VectorSubcoreMesh(core_axis_name="c", subcore_axis_name="s")

    @functools.partial(
        pl.kernel,
        mesh=mesh,
        compiler_params=pltpu.CompilerParams(
            use_tc_tiling_on_sc=False, needs_layout_passes=False
        ),
        out_type=jax.ShapeDtypeStruct((MAX_LEN, DIM, BATCH), jnp.float32),
        scratch_types=[
            pltpu.VMEM((LPG, BB), jnp.int32),
            pltpu.VMEM((QT, PDIM), jnp.float32),
            pltpu.VMEM((QT, PDIM), jnp.float32),
            pltpu.VMEM((DIM, BB + 1), jnp.float32),
            pltpu.VMEM((DIM, BB + 1), jnp.float32),
            pltpu.VMEM((LPG * DIM,), jnp.float32),
            pltpu.SemaphoreType.DMA,
            pltpu.SemaphoreType.DMA,
            pltpu.SemaphoreType.DMA,
            pltpu.SemaphoreType.DMA,
        ],
    )
    def k(idx_hbm, table_hbm, pe_hbm, out_hbm,
          idx_v, g0, g1, t0, t1, pe_v, gs0, gs1, os0, os1):
        gbufs, tbufs = (g0, g1), (t0, t1)
        gss, oss = (gs0, gs1), (os0, os1)
        w = lax.axis_index("s") * NC + lax.axis_index("c")
        grp = w // NBB
        b0 = (w % NBB) * BB
        pltpu.sync_copy(idx_hbm.at[w], idx_v)
        pltpu.sync_copy(pe_hbm.at[grp], pe_v)

        def g_start(li, q, qb):
            pltpu.async_copy(
                table_hbm.at[idx_v.at[li, pl.ds(q * QT, QT)]],
                gbufs[qb], gss[qb],
            )

        def g_wait(li, q, qb):
            pltpu.make_async_copy(
                table_hbm.at[idx_v.at[li, pl.ds(q * QT, QT)]],
                gbufs[qb], gss[qb],
            ).wait()

        g_start(0, 0, 0)
        g_start(0, 1, 1)
        d_iota = lax.iota(jnp.int32, LANES)
        zeros16 = jnp.zeros((LANES,), jnp.int32)
        d_rows = [d_iota + db * LANES for db in range(DBLKS)]

        def pair(p, carry):
            for j in range(2):
                li = 2 * p + j
                l = grp * LPG + li

                @pl.when(li >= 2)
                def _():
                    pltpu.make_async_copy(
                        tbufs[j].at[:, pl.ds(0, BB)],
                        out_hbm.at[l - 2, :, pl.ds(b0, BB)], oss[j]
                    ).wait()

                pe_vecs = [
                    pe_v[pl.ds(li * DIM + db * LANES, LANES)]
                    for db in range(DBLKS)
                ]

                for q in range(NQ):
                    qb = q % 2
                    g_wait(li, q, qb)

                    @plsc.parallel_loop(0, QT, unroll=8)
                    def _tr(r, j=j, q=q, qb=qb, pe_vecs=pe_vecs):
                        cols = zeros16 + (q * QT + r)
                        for db in range(DBLKS):
                            v = (gbufs[qb][r, pl.ds(db * LANES, LANES)]
                                 + pe_vecs[db])
                            plsc.store_scatter(
                                tbufs[j], [d_rows[db], cols], v
                            )

                    if q < 2:
                        g_start(li, q + 2, qb)
                    else:
                        @pl.when(li + 1 < LPG)
                        def _(li=li, q=q, qb=qb):
                            g_start(li + 1, q - 2, qb)

                pltpu.async_copy(
                    tbufs[j].at[:, pl.ds(0, BB)],
                    out_hbm.at[l, :, pl.ds(b0, BB)], oss[j]
                )
            return carry

        lax.fori_loop(0, LPG // 2, pair, 0)

        last = grp * LPG + LPG
        pltpu.make_async_copy(
            t0.at[:, pl.ds(0, BB)], out_hbm.at[last - 2, :, pl.ds(b0, BB)], os0
        ).wait()
        pltpu.make_async_copy(
            t1.at[:, pl.ds(0, BB)], out_hbm.at[last - 1, :, pl.ds(b0, BB)], os1
        ).wait()

    return k


_SC_KERNEL = _make_sc_kernel()


def kernel(batch, table):
    idx4 = (
        jnp.transpose(batch.astype(jnp.int32), (1, 0))
        .reshape(NLG, LPG, NBB, BB)
        .transpose(0, 2, 1, 3)
        .reshape(NW, LPG, BB)
    )
    table_p = jnp.pad(table, ((0, 0), (0, PDIM - DIM)))
    pe = jnp.asarray(_pos_encoding()).reshape(NLG, LPG * DIM)
    out_t = _SC_KERNEL(idx4, table_p, pe)
    return jnp.transpose(out_t, (2, 0, 1))
